# baseline (speedup 1.0000x reference)
import functools

import jax
import jax.numpy as jnp
from jax import lax
from jax.experimental import pallas as pl
from jax.experimental.pallas import tpu as pltpu
from jax.experimental.pallas import tpu_sc as plsc

N_NODES = 10000
N_PAD = 10240
N_EDGES = 320000
D = 128

NC = 2
NS = 16
NW = NC * NS

E_PER_TILE = N_EDGES // NW
E_C = 80
N_CHUNK = E_PER_TILE // E_C
NB = 3
NI = 6
N_SEXT = N_CHUNK // NI
ROWS_PER_TILE = N_PAD // NS
STG = E_C
NSTG = ROWS_PER_TILE // STG


def _edge_body(ei_hbm, h_hbm, agg_hbm, deg_hbm,
               idx_v, rows_v, hist_v, agg_sh,
               sem_i0, sem_i1, sem_i2, sem_i3, sem_i4, sem_i5,
               sem_g0, sem_g1, sem_g2):
    cid = lax.axis_index("c")
    sid = lax.axis_index("s")
    wid = cid * NS + sid

    zeros16 = jnp.zeros((16,), jnp.float32)
    ones16 = jnp.ones((16,), jnp.float32)

    my_ei = ei_hbm.at[wid]
    sem_i = (sem_i0, sem_i1, sem_i2, sem_i3, sem_i4, sem_i5)
    sem_g = (sem_g0, sem_g1, sem_g2)

    for k in range(NI):
        pltpu.async_copy(my_ei.at[k], idx_v.at[k], sem_i[k])

    def _z_hist(i, carry):
        hist_v[pl.ds(i * 16, 16)] = zeros16
        return carry
    lax.fori_loop(0, N_PAD // 16, _z_hist, 0)

    for c in range(2):
        pltpu.make_async_copy(my_ei.at[c], idx_v.at[c], sem_i[c]).wait()
        pltpu.async_copy(h_hbm.at[idx_v.at[c].at[0]], rows_v.at[c],
                         sem_g[c])

    def _z_stg(i, carry):
        for j in range(D // 16):
            rows_v[2, i, pl.ds(j * 16, 16)] = zeros16
        return carry
    lax.fori_loop(0, STG, _z_stg, 0)

    row0 = sid * ROWS_PER_TILE
    for k in range(NSTG):
        pltpu.sync_copy(rows_v.at[2], agg_sh.at[pl.ds(row0 + k * STG, STG)])

    pltpu.make_async_copy(my_ei.at[2], idx_v.at[2], sem_i[2]).wait()
    pltpu.async_copy(h_hbm.at[idx_v.at[2].at[0]], rows_v.at[2], sem_g[2])

    plsc.subcore_barrier()

    def _count(s):
        for k in range(E_C // 16):
            idx = idx_v[s, 1, pl.ds(k * 16, 16)]
            plsc.addupdate_scatter(hist_v, [idx], ones16)

    def _chunk_step(c, s, r, prefetch_idx, prefetch_gather):
        pltpu.make_async_copy(h_hbm.at[idx_v.at[s].at[0]],
                              rows_v.at[r], sem_g[r]).wait()
        pltpu.sync_copy(rows_v.at[r], agg_sh.at[idx_v.at[s].at[1]],
                        add=True)
        _count(s)
        if prefetch_idx:
            @pl.when(c + NI < N_CHUNK)
            def _():
                pltpu.async_copy(my_ei.at[c + NI], idx_v.at[s], sem_i[s])
        if prefetch_gather:
            s3 = (s + NB) % NI
            pltpu.make_async_copy(my_ei.at[c + NB], idx_v.at[s3],
                                  sem_i[s3]).wait()
            pltpu.async_copy(h_hbm.at[idx_v.at[s3].at[0]], rows_v.at[r],
                             sem_g[r])

    def _sextet(t, carry):
        c0 = t * NI
        for s in range(NI):
            _chunk_step(c0 + s, s, s % NB, True, True)
        return carry

    lax.fori_loop(0, N_SEXT, _sextet, 0)

    for s in range(N_CHUNK - N_SEXT * NI):
        c = N_SEXT * NI + s
        _chunk_step(c, s, s % NB, False, c + NB < N_CHUNK)

    plsc.subcore_barrier()

    pltpu.async_copy(hist_v, deg_hbm.at[cid].at[sid], sem_i[0])
    for k in range(NSTG):
        b = k % 2
        r = row0 + k * STG
        if k >= 2:
            pltpu.make_async_copy(
                rows_v.at[b], agg_hbm.at[cid].at[pl.ds(r, STG)],
                sem_g[b]).wait()
        pltpu.sync_copy(agg_sh.at[pl.ds(r, STG)], rows_v.at[b])
        pltpu.async_copy(rows_v.at[b], agg_hbm.at[cid].at[pl.ds(r, STG)],
                         sem_g[b])
    for b in range(2):
        pltpu.make_async_copy(rows_v.at[b],
                              agg_hbm.at[cid].at[pl.ds(row0, STG)],
                              sem_g[b]).wait()
    pltpu.make_async_copy(hist_v, deg_hbm.at[cid].at[sid], sem_i[0]).wait()


def _combine_body(wt_ref, b_ref, agg_ref, deg_ref, o_ref):
    a = agg_ref[0] + agg_ref[1]
    d = jnp.sum(deg_ref[...], axis=(0, 1))[:, None]
    mean = a / jnp.maximum(d, 1.0)
    mask = jnp.where(d > 0.0, 1.0, 0.0)
    o_ref[...] = (jnp.dot(mean, wt_ref[...],
                          preferred_element_type=jnp.float32)
                  + mask * b_ref[...])


def kernel(h, edge_index, W, b):
    ei = edge_index.astype(jnp.int32).reshape(2, NW, N_CHUNK, E_C)
    ei = jnp.stack([ei[0], ei[1]], axis=2)

    mesh = plsc.VectorSubcoreMesh(core_axis_name="c", subcore_axis_name="s",
                                  num_cores=NC, num_subcores=NS)
    edge_kernel = functools.partial(
        pl.kernel,
        mesh=mesh,
        out_type=(jax.ShapeDtypeStruct((NC, N_PAD, D), jnp.float32),
                  jax.ShapeDtypeStruct((NC, NS, N_PAD), jnp.float32)),
        scratch_types=[
            pltpu.VMEM((NI, 2, E_C), jnp.int32),
            pltpu.VMEM((NB, E_C, D), jnp.float32),
            pltpu.VMEM((N_PAD,), jnp.float32),
            pltpu.VMEM_SHARED((N_PAD, D), jnp.float32),
            pltpu.SemaphoreType.DMA,
            pltpu.SemaphoreType.DMA,
            pltpu.SemaphoreType.DMA,
            pltpu.SemaphoreType.DMA,
            pltpu.SemaphoreType.DMA,
            pltpu.SemaphoreType.DMA,
            pltpu.SemaphoreType.DMA,
            pltpu.SemaphoreType.DMA,
            pltpu.SemaphoreType.DMA,
        ],
        compiler_params=pltpu.CompilerParams(use_tc_tiling_on_sc=False,
                                             needs_layout_passes=False),
    )(_edge_body)
    agg_p, deg_p = edge_kernel(ei, h)

    BLK = 1024
    out = pl.pallas_call(
        _combine_body,
        grid=(N_PAD // BLK,),
        in_specs=[
            pl.BlockSpec((D, D), lambda i: (0, 0)),
            pl.BlockSpec((1, D), lambda i: (0, 0)),
            pl.BlockSpec((NC, BLK, D), lambda i: (0, i, 0)),
            pl.BlockSpec((NC, NS, BLK), lambda i: (0, 0, i)),
        ],
        out_specs=pl.BlockSpec((BLK, D), lambda i: (i, 0)),
        out_shape=jax.ShapeDtypeStruct((N_PAD, D), jnp.float32),
    )(W.T, b.reshape(1, D), agg_p, deg_p)
    return out[:N_NODES]


# speedup vs baseline: 1.0919x; 1.0919x over previous
import functools

import jax
import jax.numpy as jnp
from jax import lax
from jax.experimental import pallas as pl
from jax.experimental.pallas import tpu as pltpu
from jax.experimental.pallas import tpu_sc as plsc

N_NODES = 10000
N_PAD = 10240
N_EDGES = 320000
D = 128

NC = 2
NS = 16
NW = NC * NS

E_PER_TILE = N_EDGES // NW
E_C = 80
N_CHUNK = E_PER_TILE // E_C
N_PAIR = N_CHUNK // 2
ROWS_PER_TILE = N_PAD // NS
STG = E_C
NSTG = ROWS_PER_TILE // STG


def _edge_body(ei_hbm, h_hbm, agg_hbm, deg_hbm,
               src_v, dstb_v, rows_v, hist_v, agg_sh,
               sem_g0, sem_g1, sem_d0, sem_d1):
    cid = lax.axis_index("c")
    sid = lax.axis_index("s")
    wid = cid * NS + sid

    zeros16 = jnp.zeros((16,), jnp.float32)
    ones16 = jnp.ones((16,), jnp.float32)

    pltpu.sync_copy(ei_hbm.at[0].at[wid], src_v)
    my_dst = ei_hbm.at[1].at[wid]

    pltpu.async_copy(my_dst.at[0], dstb_v.at[0], sem_d0)
    pltpu.async_copy(my_dst.at[1], dstb_v.at[1], sem_d1)
    pltpu.async_copy(h_hbm.at[src_v.at[0]], rows_v.at[0], sem_g0)

    def _z_hist(i, carry):
        hist_v[pl.ds(i * 16, 16)] = zeros16
        return carry
    lax.fori_loop(0, N_PAD // 16, _z_hist, 0)

    def _z_stg(i, carry):
        for j in range(D // 16):
            rows_v[1, i, pl.ds(j * 16, 16)] = zeros16
        return carry
    lax.fori_loop(0, STG, _z_stg, 0)

    row0 = sid * ROWS_PER_TILE
    for k in range(NSTG):
        pltpu.async_copy(rows_v.at[1], agg_sh.at[pl.ds(row0 + k * STG, STG)],
                         sem_g1)
    for k in range(NSTG):
        pltpu.make_async_copy(rows_v.at[1],
                              agg_sh.at[pl.ds(row0 + k * STG, STG)],
                              sem_g1).wait()

    pltpu.async_copy(h_hbm.at[src_v.at[1]], rows_v.at[1], sem_g1)

    plsc.subcore_barrier()

    def _count(q):
        for k in range(E_C // 16):
            idx = dstb_v[q, pl.ds(k * 16, 16)]
            plsc.addupdate_scatter(hist_v, [idx], ones16)

    def _pair(p, carry):
        c0 = 2 * p
        c1 = 2 * p + 1

        pltpu.make_async_copy(h_hbm.at[src_v.at[c0]],
                              rows_v.at[0], sem_g0).wait()
        pltpu.make_async_copy(my_dst.at[c0], dstb_v.at[0], sem_d0).wait()
        pltpu.sync_copy(rows_v.at[0], agg_sh.at[dstb_v.at[0]], add=True)
        pltpu.async_copy(h_hbm.at[src_v.at[c0 + 2]], rows_v.at[0], sem_g0)
        _count(0)
        pltpu.async_copy(my_dst.at[c0 + 2], dstb_v.at[0], sem_d0)

        pltpu.make_async_copy(h_hbm.at[src_v.at[c1]],
                              rows_v.at[1], sem_g1).wait()
        pltpu.make_async_copy(my_dst.at[c1], dstb_v.at[1], sem_d1).wait()
        pltpu.sync_copy(rows_v.at[1], agg_sh.at[dstb_v.at[1]], add=True)

        @pl.when(c1 + 2 < N_CHUNK)
        def _():
            pltpu.async_copy(h_hbm.at[src_v.at[c1 + 2]], rows_v.at[1], sem_g1)
        _count(1)

        @pl.when(c1 + 2 < N_CHUNK)
        def _():
            pltpu.async_copy(my_dst.at[c1 + 2], dstb_v.at[1], sem_d1)

        return carry

    lax.fori_loop(0, N_PAIR, _pair, 0)

    c_t = N_CHUNK - 1
    pltpu.make_async_copy(h_hbm.at[src_v.at[c_t]],
                          rows_v.at[0], sem_g0).wait()
    pltpu.make_async_copy(my_dst.at[c_t], dstb_v.at[0], sem_d0).wait()
    pltpu.sync_copy(rows_v.at[0], agg_sh.at[dstb_v.at[0]], add=True)
    _count(0)

    plsc.subcore_barrier()

    pltpu.async_copy(hist_v, deg_hbm.at[cid].at[sid], sem_d0)
    wsem = (sem_g0, sem_g1)
    for k in range(NSTG):
        b = k % 2
        r = row0 + k * STG
        if k >= 2:
            pltpu.make_async_copy(
                rows_v.at[b], agg_hbm.at[cid].at[pl.ds(r, STG)],
                wsem[b]).wait()
        pltpu.sync_copy(agg_sh.at[pl.ds(r, STG)], rows_v.at[b])
        pltpu.async_copy(rows_v.at[b], agg_hbm.at[cid].at[pl.ds(r, STG)],
                         wsem[b])
    for b in range(2):
        pltpu.make_async_copy(rows_v.at[b],
                              agg_hbm.at[cid].at[pl.ds(row0, STG)],
                              wsem[b]).wait()
    pltpu.make_async_copy(hist_v, deg_hbm.at[cid].at[sid], sem_d0).wait()


def _combine_body(wt_ref, b_ref, agg_ref, deg_ref, o_ref):
    a = agg_ref[0] + agg_ref[1]
    d = jnp.sum(deg_ref[...], axis=(0, 1))[:, None]
    mean = a / jnp.maximum(d, 1.0)
    mask = jnp.where(d > 0.0, 1.0, 0.0)
    o_ref[...] = (jnp.dot(mean, wt_ref[...],
                          preferred_element_type=jnp.float32)
                  + mask * b_ref[...])


def kernel(h, edge_index, W, b):
    ei = edge_index.astype(jnp.int32).reshape(2, NW, N_CHUNK, E_C)

    mesh = plsc.VectorSubcoreMesh(core_axis_name="c", subcore_axis_name="s",
                                  num_cores=NC, num_subcores=NS)
    edge_kernel = functools.partial(
        pl.kernel,
        mesh=mesh,
        out_type=(jax.ShapeDtypeStruct((NC, N_PAD, D), jnp.float32),
                  jax.ShapeDtypeStruct((NC, NS, N_PAD), jnp.float32)),
        scratch_types=[
            pltpu.VMEM((N_CHUNK, E_C), jnp.int32),
            pltpu.VMEM((2, E_C), jnp.int32),
            pltpu.VMEM((2, E_C, D), jnp.float32),
            pltpu.VMEM((N_PAD,), jnp.float32),
            pltpu.VMEM_SHARED((N_PAD, D), jnp.float32),
            pltpu.SemaphoreType.DMA,
            pltpu.SemaphoreType.DMA,
            pltpu.SemaphoreType.DMA,
            pltpu.SemaphoreType.DMA,
        ],
        compiler_params=pltpu.CompilerParams(use_tc_tiling_on_sc=False,
                                             needs_layout_passes=False),
    )(_edge_body)
    agg_p, deg_p = edge_kernel(ei, h)

    BLK = 1024
    out = pl.pallas_call(
        _combine_body,
        grid=(N_PAD // BLK,),
        in_specs=[
            pl.BlockSpec((D, D), lambda i: (0, 0)),
            pl.BlockSpec((1, D), lambda i: (0, 0)),
            pl.BlockSpec((NC, BLK, D), lambda i: (0, i, 0)),
            pl.BlockSpec((NC, NS, BLK), lambda i: (0, 0, i)),
        ],
        out_specs=pl.BlockSpec((BLK, D), lambda i: (i, 0)),
        out_shape=jax.ShapeDtypeStruct((N_PAD, D), jnp.float32),
    )(W.T, b.reshape(1, D), agg_p, deg_p)
    return out[:N_NODES]
